# baseline (speedup 1.0000x reference)
import functools

import jax
import jax.numpy as jnp
from jax import lax
from jax.experimental import pallas as pl
from jax.experimental.pallas import tpu as pltpu
from jax.experimental.pallas import tpu_sc as plsc

_LMAX = 2
_C = 128
_MID = 32
_N = 10000
_K = 6
_BASIS = 9
_D = _BASIS * _C

_E = _N * _K
_NW = 16
_N_PAD = 10240
_E_PAD = _N_PAD * _K
_NH = _N_PAD // 2
_EH = _NH * _K
_NODES_W = _NH // _NW
_B_PER_W = _EH // _NW
_NGRP = _B_PER_W // 16
_NCHN = 4
_CH = _NCHN * _K
_NCH = _NODES_W // _NCHN

_NB_A = 320
_NB_B = 80


def _rsqrt_nr(s):
    i = plsc.bitcast(s, jnp.int32)
    i = 0x5F3759DF - lax.shift_right_logical(i, 1)
    y = plsc.bitcast(i, jnp.float32)
    for _ in range(3):
        y = y * (1.5 - 0.5 * s * y * y)
    return y


def _sc_gather_reduce(feats2d, px, py, pz, idx_half, dst_half):
    nc = 1
    mesh = plsc.VectorSubcoreMesh(core_axis_name="c", subcore_axis_name="s",
                                  num_cores=1)

    @functools.partial(
        pl.kernel,
        mesh=mesh,
        out_type=jax.ShapeDtypeStruct((_NH, _D), jnp.float32),
        scratch_types=[
            pltpu.VMEM((_B_PER_W,), jnp.int32),
            pltpu.VMEM((_B_PER_W,), jnp.int32),
            pltpu.VMEM((_CH, _D), jnp.float32),
            pltpu.VMEM((_CH, _D), jnp.float32),
            pltpu.VMEM((2 * _NCHN, _D), jnp.float32),
            pltpu.VMEM((_N_PAD,), jnp.float32),
            pltpu.VMEM((_N_PAD,), jnp.float32),
            pltpu.VMEM((_N_PAD,), jnp.float32),
            pltpu.VMEM((_B_PER_W * 16,), jnp.float32),
            pltpu.SemaphoreType.DMA,
            pltpu.SemaphoreType.DMA,
        ],
        compiler_params=pltpu.CompilerParams(needs_layout_passes=False),
    )
    def k(feats_hbm, px_hbm, py_hbm, pz_hbm, idx_hbm, dst_hbm, x_hbm,
          idx_v, dst_v, rows_a, rows_b, out_a, px_v, py_v, pz_v,
          sh_v, gsem_a, gsem_b):
        wid = lax.axis_index("s") * nc + lax.axis_index("c")
        ebase = wid * _B_PER_W
        nbase = wid * _NODES_W
        pltpu.sync_copy(idx_hbm.at[pl.ds(ebase, _B_PER_W)], idx_v)
        pltpu.sync_copy(dst_hbm.at[pl.ds(ebase, _B_PER_W)], dst_v)
        pltpu.sync_copy(px_hbm, px_v)
        pltpu.sync_copy(py_hbm, py_v)
        pltpu.sync_copy(pz_hbm, pz_v)

        inv_k = 1.0 / _K
        s3 = 1.7320508075688772

        def pbody(g, carry):
            i16 = idx_v[pl.ds(g * 16, 16)]
            d16 = dst_v[pl.ds(g * 16, 16)]
            nx = plsc.load_gather(px_v, [i16])
            ny = plsc.load_gather(py_v, [i16])
            nz = plsc.load_gather(pz_v, [i16])
            ox = plsc.load_gather(px_v, [d16])
            oy = plsc.load_gather(py_v, [d16])
            oz = plsc.load_gather(pz_v, [d16])
            rx = nx - ox
            ry = ny - oy
            rz = nz - oz
            s = rx * rx + ry * ry + rz * rz + 1e-12
            ir = _rsqrt_nr(s)
            ux = rx * ir
            uy = ry * ir
            uz = rz * ir
            sh = (
                jnp.full((16,), inv_k, jnp.float32),
                ux * inv_k, uy * inv_k, uz * inv_k,
                (s3 * inv_k) * ux * uy,
                (s3 * inv_k) * uy * uz,
                inv_k * (1.5 * uz * uz - 0.5),
                (s3 * inv_k) * ux * uz,
                (s3 * 0.5 * inv_k) * (ux * ux - uy * uy),
            )
            lanes16 = lax.iota(jnp.int32, 16) * 16 + g * 256
            for b in range(_BASIS):
                plsc.store_scatter(sh_v, [lanes16 + b], sh[b])
            return carry

        lax.fori_loop(0, _NGRP, pbody, 0)

        def reduce_chunk(c, buf, stage):
            def node_body(nl, carry2):
                e0 = nl * _K
                ebase_c = c * _CH + e0
                shrows = [sh_v[pl.ds((ebase_c + kk) * 16, 16)]
                          for kk in range(_K)]
                for b in range(_BASIS):
                    col = b * _C
                    acc = [None] * 8
                    for kk in range(_K):
                        w = shrows[kk][b]
                        for j in range(8):
                            r = buf[e0 + kk, pl.ds(col + j * 16, 16)]
                            acc[j] = w * r if kk == 0 else acc[j] + w * r
                    for j in range(8):
                        out_a[stage * _NCHN + nl,
                              pl.ds(col + j * 16, 16)] = acc[j]
                return carry2

            lax.fori_loop(0, _NCHN, node_body, 0)

        def gather_into(c, buf, sem):
            idx_c = idx_v.at[pl.ds(c * _CH, _CH)]
            return pltpu.async_copy(feats_hbm.at[idx_c], buf, sem)

        gather_into(0, rows_a, gsem_a).wait()

        def chunk_body(c, carry):
            nxt = lax.rem(c + 1, _NCH)

            @pl.when(lax.rem(c, 2) == 0)
            def _():
                d = gather_into(nxt, rows_b, gsem_b)
                reduce_chunk(c, rows_a, 0)
                d.wait()

            @pl.when(lax.rem(c, 2) == 1)
            def _():
                d = gather_into(nxt, rows_a, gsem_a)
                reduce_chunk(c, rows_b, 1)
                off = pl.multiple_of(nbase + (c - 1) * _NCHN, 2 * _NCHN)
                pltpu.sync_copy(out_a, x_hbm.at[pl.ds(off, 2 * _NCHN)])
                d.wait()

            return carry

        lax.fori_loop(0, _NCH, chunk_body, 0)

    return k(feats2d, px, py, pz, idx_half, dst_half)


def _tc_body(x_ref, sf_ref, wcg_ref, lnw_ref, lnb_ref, w1t_ref, wgt_ref,
             w2t_ref, out_ref):
    wcg = wcg_ref[...]
    xs = []
    for b in range(_BASIS):
        xb = jnp.dot(x_ref[:, pl.ds(b * _C, _C)], wcg,
                     preferred_element_type=jnp.float32)
        xs.append(xb + sf_ref[:, pl.ds(b * _C, _C)])

    eps = 1e-5
    ys = [None] * _BASIS
    mu = jnp.mean(xs[0], axis=1, keepdims=True)
    bc = xs[0] - mu
    var0 = jnp.mean(bc * bc, axis=1, keepdims=True)
    ys[0] = bc * lax.rsqrt(var0 + eps) * lnw_ref[0:1, :] + lnb_ref[0:1, :]
    var1 = (jnp.mean(xs[1] * xs[1], axis=1, keepdims=True)
            + jnp.mean(xs[2] * xs[2], axis=1, keepdims=True)
            + jnp.mean(xs[3] * xs[3], axis=1, keepdims=True)) * (1.0 / 3.0)
    sc1 = lax.rsqrt(var1 + eps)
    for b in range(1, 4):
        ys[b] = xs[b] * sc1 * lnw_ref[1:2, :]
    var2 = (jnp.mean(xs[4] * xs[4], axis=1, keepdims=True)
            + jnp.mean(xs[5] * xs[5], axis=1, keepdims=True)
            + jnp.mean(xs[6] * xs[6], axis=1, keepdims=True)
            + jnp.mean(xs[7] * xs[7], axis=1, keepdims=True)
            + jnp.mean(xs[8] * xs[8], axis=1, keepdims=True)) * 0.2
    sc2 = lax.rsqrt(var2 + eps)
    for b in range(4, 9):
        ys[b] = xs[b] * sc2 * lnw_ref[2:3, :]

    w1t = w1t_ref[...]
    hs = [jnp.dot(ys[b], w1t, preferred_element_type=jnp.float32)
          for b in range(_BASIS)]

    after = jnp.dot(hs[0], wgt_ref[...], preferred_element_type=jnp.float32)
    type0 = after[:, 0:_MID]
    mult = jax.nn.sigmoid(after[:, _MID:])
    gs = [type0 * jax.nn.sigmoid(type0)]
    for b in range(1, 4):
        gs.append(hs[b] * mult[:, 0:_MID])
    for b in range(4, 9):
        gs.append(hs[b] * mult[:, _MID:2 * _MID])

    w2t = w2t_ref[...]
    for b in range(_BASIS):
        ob = jnp.dot(gs[b], w2t, preferred_element_type=jnp.float32)
        out_ref[:, pl.ds(b * _C, _C)] = ob + xs[b]


def _tc_dense(xsum, feats2d, W_cg, lnw, lnb, W1t, Wgt, W2t,
              nb, grid, sf_off):
    return pl.pallas_call(
        _tc_body,
        grid=(grid,),
        in_specs=[
            pl.BlockSpec((nb, _D), lambda i: (i, 0)),
            pl.BlockSpec((nb, _D), lambda i, o=sf_off: (i + o, 0)),
            pl.BlockSpec((_C, _C), lambda i: (0, 0)),
            pl.BlockSpec((_LMAX + 1, _C), lambda i: (0, 0)),
            pl.BlockSpec((1, _C), lambda i: (0, 0)),
            pl.BlockSpec((_C, _MID), lambda i: (0, 0)),
            pl.BlockSpec((_MID, _MID * (_LMAX + 1)), lambda i: (0, 0)),
            pl.BlockSpec((_MID, _C), lambda i: (0, 0)),
        ],
        out_specs=pl.BlockSpec((nb, _D), lambda i: (i, 0)),
        out_shape=jax.ShapeDtypeStruct((nb * grid, _D), jnp.float32),
        compiler_params=pltpu.CompilerParams(
            dimension_semantics=("arbitrary",),
        ),
    )(xsum, feats2d, W_cg, lnw, lnb, W1t, Wgt, W2t)


def kernel(s_feats, s_points, neighbor_indices, W_cg, ln_weight, ln_bias,
           W1, Wg, W2):
    feats2d = s_feats.reshape(_N, _D)
    ppad = jnp.pad(s_points, ((0, _N_PAD - _N), (0, 0)))
    idx_pad = jnp.pad(neighbor_indices.reshape(-1).astype(jnp.int32),
                      (0, _E_PAD - _E))
    dst_pad = jnp.repeat(
        lax.iota(jnp.int32, _N_PAD), _K, total_repeat_length=_E_PAD)

    px, py, pz = ppad[:, 0], ppad[:, 1], ppad[:, 2]
    xa = _sc_gather_reduce(feats2d, px, py, pz,
                           idx_pad[:_EH], dst_pad[:_EH])
    xb = _sc_gather_reduce(feats2d, px, py, pz,
                           idx_pad[_EH:], dst_pad[_EH:])
    lnb2 = ln_bias.reshape(1, _C)
    out_a = _tc_dense(xa, feats2d, W_cg, ln_weight, lnb2, W1.T, Wg.T, W2.T,
                      _NB_A, _NH // _NB_A, 0)
    out_b = _tc_dense(xb, feats2d, W_cg, ln_weight, lnb2, W1.T, Wg.T, W2.T,
                      _NB_B, (_N - _NH) // _NB_B, _NH // _NB_B)
    out = jnp.concatenate([out_a, out_b], axis=0)
    return out.reshape(_N, _BASIS, _C)


# speedup vs baseline: 1.3111x; 1.3111x over previous
import functools

import jax
import jax.numpy as jnp
from jax import lax
from jax.experimental import pallas as pl
from jax.experimental.pallas import tpu as pltpu
from jax.experimental.pallas import tpu_sc as plsc

_LMAX = 2
_C = 128
_MID = 32
_N = 10000
_K = 6
_BASIS = 9
_D = _BASIS * _C

_E = _N * _K
_NW = 32
_N_PAD = 10240
_E_PAD = _N_PAD * _K
_NODES_W = _N_PAD // _NW
_B_PER_W = _E_PAD // _NW
_NGRP = _B_PER_W // 16
_NCHN = 4
_CH = _NCHN * _K
_NCH = _NODES_W // _NCHN

_NB = 400
_GRID = _N // _NB


def _rsqrt_nr(s):
    i = plsc.bitcast(s, jnp.int32)
    i = 0x5F3759DF - lax.shift_right_logical(i, 1)
    y = plsc.bitcast(i, jnp.float32)
    for _ in range(3):
        y = y * (1.5 - 0.5 * s * y * y)
    return y


def _sc_gather_reduce(feats2d, px, py, pz, idx_pad, dst_pad):
    info = plsc.get_sparse_core_info()
    nc = info.num_cores

    mesh = plsc.VectorSubcoreMesh(core_axis_name="c", subcore_axis_name="s")

    @functools.partial(
        pl.kernel,
        mesh=mesh,
        out_type=jax.ShapeDtypeStruct((_N_PAD, _D), jnp.float32),
        scratch_types=[
            pltpu.VMEM((_B_PER_W,), jnp.int32),
            pltpu.VMEM((_B_PER_W,), jnp.int32),
            pltpu.VMEM((_CH, _D), jnp.float32),
            pltpu.VMEM((_CH, _D), jnp.float32),
            pltpu.VMEM((2 * _NCHN, _D), jnp.float32),
            pltpu.VMEM((_N_PAD,), jnp.float32),
            pltpu.VMEM((_N_PAD,), jnp.float32),
            pltpu.VMEM((_N_PAD,), jnp.float32),
            pltpu.VMEM((_B_PER_W * 16,), jnp.float32),
            pltpu.SemaphoreType.DMA,
            pltpu.SemaphoreType.DMA,
        ],
        compiler_params=pltpu.CompilerParams(needs_layout_passes=False),
    )
    def k(feats_hbm, px_hbm, py_hbm, pz_hbm, idx_hbm, dst_hbm, x_hbm,
          idx_v, dst_v, rows_a, rows_b, out_a, px_v, py_v, pz_v,
          sh_v, gsem_a, gsem_b):
        wid = lax.axis_index("s") * nc + lax.axis_index("c")
        ebase = wid * _B_PER_W
        nbase = wid * _NODES_W
        pltpu.sync_copy(idx_hbm.at[pl.ds(ebase, _B_PER_W)], idx_v)
        pltpu.sync_copy(dst_hbm.at[pl.ds(ebase, _B_PER_W)], dst_v)
        pltpu.sync_copy(px_hbm, px_v)
        pltpu.sync_copy(py_hbm, py_v)
        pltpu.sync_copy(pz_hbm, pz_v)

        inv_k = 1.0 / _K
        s3 = 1.7320508075688772

        @plsc.parallel_loop(0, _NGRP)
        def pbody(g):
            i16 = idx_v[pl.ds(g * 16, 16)]
            d16 = dst_v[pl.ds(g * 16, 16)]
            nx = plsc.load_gather(px_v, [i16])
            ny = plsc.load_gather(py_v, [i16])
            nz = plsc.load_gather(pz_v, [i16])
            ox = plsc.load_gather(px_v, [d16])
            oy = plsc.load_gather(py_v, [d16])
            oz = plsc.load_gather(pz_v, [d16])
            rx = nx - ox
            ry = ny - oy
            rz = nz - oz
            s = rx * rx + ry * ry + rz * rz + 1e-12
            ir = _rsqrt_nr(s)
            ux = rx * ir
            uy = ry * ir
            uz = rz * ir
            sh = (
                jnp.full((16,), inv_k, jnp.float32),
                ux * inv_k, uy * inv_k, uz * inv_k,
                (s3 * inv_k) * ux * uy,
                (s3 * inv_k) * uy * uz,
                inv_k * (1.5 * uz * uz - 0.5),
                (s3 * inv_k) * ux * uz,
                (s3 * 0.5 * inv_k) * (ux * ux - uy * uy),
            )
            lanes16 = lax.iota(jnp.int32, 16) * 16 + g * 256
            for b in range(_BASIS):
                plsc.store_scatter(sh_v, [lanes16 + b], sh[b])

        def reduce_chunk(c, buf, stage):
            @plsc.parallel_loop(0, _NCHN)
            def node_body(nl):
                e0 = nl * _K
                ebase_c = c * _CH + e0
                shrows = [sh_v[pl.ds((ebase_c + kk) * 16, 16)]
                          for kk in range(_K)]
                for b in range(_BASIS):
                    col = b * _C
                    acc = [None] * 8
                    for kk in range(_K):
                        w = shrows[kk][b]
                        for j in range(8):
                            r = buf[e0 + kk, pl.ds(col + j * 16, 16)]
                            acc[j] = w * r if kk == 0 else acc[j] + w * r
                    for j in range(8):
                        out_a[stage * _NCHN + nl,
                              pl.ds(col + j * 16, 16)] = acc[j]

        def gather_into(c, buf, sem):
            idx_c = idx_v.at[pl.ds(c * _CH, _CH)]
            return pltpu.async_copy(feats_hbm.at[idx_c], buf, sem)

        gather_into(0, rows_a, gsem_a).wait()

        def chunk_body(c, carry):
            nxt = lax.rem(c + 1, _NCH)

            @pl.when(lax.rem(c, 2) == 0)
            def _():
                d = gather_into(nxt, rows_b, gsem_b)
                reduce_chunk(c, rows_a, 0)
                d.wait()

            @pl.when(lax.rem(c, 2) == 1)
            def _():
                d = gather_into(nxt, rows_a, gsem_a)
                reduce_chunk(c, rows_b, 1)
                off = pl.multiple_of(nbase + (c - 1) * _NCHN, 2 * _NCHN)
                pltpu.sync_copy(out_a, x_hbm.at[pl.ds(off, 2 * _NCHN)])
                d.wait()

            return carry

        lax.fori_loop(0, _NCH, chunk_body, 0)

    return k(feats2d, px, py, pz, idx_pad, dst_pad)


def _tc_body(x_ref, sf_ref, wcg_ref, lnw_ref, lnb_ref, w1t_ref, wgt_ref,
             w2t_ref, out_ref):
    wcg = wcg_ref[...]
    xs = []
    for b in range(_BASIS):
        xb = jnp.dot(x_ref[:, pl.ds(b * _C, _C)], wcg,
                     preferred_element_type=jnp.float32)
        xs.append(xb + sf_ref[:, pl.ds(b * _C, _C)])

    eps = 1e-5
    ys = [None] * _BASIS
    mu = jnp.mean(xs[0], axis=1, keepdims=True)
    bc = xs[0] - mu
    var0 = jnp.mean(bc * bc, axis=1, keepdims=True)
    ys[0] = bc * lax.rsqrt(var0 + eps) * lnw_ref[0:1, :] + lnb_ref[0:1, :]
    var1 = (jnp.mean(xs[1] * xs[1], axis=1, keepdims=True)
            + jnp.mean(xs[2] * xs[2], axis=1, keepdims=True)
            + jnp.mean(xs[3] * xs[3], axis=1, keepdims=True)) * (1.0 / 3.0)
    sc1 = lax.rsqrt(var1 + eps)
    for b in range(1, 4):
        ys[b] = xs[b] * sc1 * lnw_ref[1:2, :]
    var2 = (jnp.mean(xs[4] * xs[4], axis=1, keepdims=True)
            + jnp.mean(xs[5] * xs[5], axis=1, keepdims=True)
            + jnp.mean(xs[6] * xs[6], axis=1, keepdims=True)
            + jnp.mean(xs[7] * xs[7], axis=1, keepdims=True)
            + jnp.mean(xs[8] * xs[8], axis=1, keepdims=True)) * 0.2
    sc2 = lax.rsqrt(var2 + eps)
    for b in range(4, 9):
        ys[b] = xs[b] * sc2 * lnw_ref[2:3, :]

    w1t = w1t_ref[...]
    hs = [jnp.dot(ys[b], w1t, preferred_element_type=jnp.float32)
          for b in range(_BASIS)]

    after = jnp.dot(hs[0], wgt_ref[...], preferred_element_type=jnp.float32)
    type0 = after[:, 0:_MID]
    mult = jax.nn.sigmoid(after[:, _MID:])
    gs = [type0 * jax.nn.sigmoid(type0)]
    for b in range(1, 4):
        gs.append(hs[b] * mult[:, 0:_MID])
    for b in range(4, 9):
        gs.append(hs[b] * mult[:, _MID:2 * _MID])

    w2t = w2t_ref[...]
    for b in range(_BASIS):
        ob = jnp.dot(gs[b], w2t, preferred_element_type=jnp.float32)
        out_ref[:, pl.ds(b * _C, _C)] = ob + xs[b]


def _tc_dense(xsum, feats2d, W_cg, lnw, lnb, W1t, Wgt, W2t):
    return pl.pallas_call(
        _tc_body,
        grid=(_GRID,),
        in_specs=[
            pl.BlockSpec((_NB, _D), lambda i: (i, 0)),
            pl.BlockSpec((_NB, _D), lambda i: (i, 0)),
            pl.BlockSpec((_C, _C), lambda i: (0, 0)),
            pl.BlockSpec((_LMAX + 1, _C), lambda i: (0, 0)),
            pl.BlockSpec((1, _C), lambda i: (0, 0)),
            pl.BlockSpec((_C, _MID), lambda i: (0, 0)),
            pl.BlockSpec((_MID, _MID * (_LMAX + 1)), lambda i: (0, 0)),
            pl.BlockSpec((_MID, _C), lambda i: (0, 0)),
        ],
        out_specs=pl.BlockSpec((_NB, _D), lambda i: (i, 0)),
        out_shape=jax.ShapeDtypeStruct((_N, _D), jnp.float32),
        compiler_params=pltpu.CompilerParams(
            dimension_semantics=("arbitrary",),
        ),
    )(xsum, feats2d, W_cg, lnw, lnb, W1t, Wgt, W2t)


def kernel(s_feats, s_points, neighbor_indices, W_cg, ln_weight, ln_bias,
           W1, Wg, W2):
    feats2d = s_feats.reshape(_N, _D)
    ppad = jnp.pad(s_points, ((0, _N_PAD - _N), (0, 0)))
    idx_pad = jnp.pad(neighbor_indices.reshape(-1).astype(jnp.int32),
                      (0, _E_PAD - _E))
    dst_pad = jnp.repeat(
        lax.iota(jnp.int32, _N_PAD), _K, total_repeat_length=_E_PAD)

    xsum = _sc_gather_reduce(feats2d, ppad[:, 0], ppad[:, 1], ppad[:, 2],
                             idx_pad, dst_pad)

    out = _tc_dense(xsum, feats2d, W_cg, ln_weight,
                    ln_bias.reshape(1, _C), W1.T, Wg.T, W2.T)
    return out.reshape(_N, _BASIS, _C)


# speedup vs baseline: 1.3132x; 1.0016x over previous
import functools

import jax
import jax.numpy as jnp
from jax import lax
from jax.experimental import pallas as pl
from jax.experimental.pallas import tpu as pltpu
from jax.experimental.pallas import tpu_sc as plsc

_LMAX = 2
_C = 128
_MID = 32
_N = 10000
_K = 6
_BASIS = 9
_D = _BASIS * _C

_E = _N * _K
_NW = 32
_N_PAD = 10240
_E_PAD = _N_PAD * _K
_NODES_W = _N_PAD // _NW
_B_PER_W = _E_PAD // _NW
_NGRP = _B_PER_W // 16
_NCHN = 4
_CH = _NCHN * _K
_NCH = _NODES_W // _NCHN

_NB = 400
_GRID = _N // _NB


def _rsqrt_nr(s):
    i = plsc.bitcast(s, jnp.int32)
    i = 0x5F3759DF - lax.shift_right_logical(i, 1)
    y = plsc.bitcast(i, jnp.float32)
    for _ in range(3):
        y = y * (1.5 - 0.5 * s * y * y)
    return y


def _sc_gather_reduce(feats2d, px, py, pz, idx_pad, dst_pad):
    info = plsc.get_sparse_core_info()
    nc = info.num_cores

    mesh = plsc.VectorSubcoreMesh(core_axis_name="c", subcore_axis_name="s")

    @functools.partial(
        pl.kernel,
        mesh=mesh,
        out_type=jax.ShapeDtypeStruct((_N_PAD, _D), jnp.float32),
        scratch_types=[
            pltpu.VMEM((_B_PER_W,), jnp.int32),
            pltpu.VMEM((_B_PER_W,), jnp.int32),
            pltpu.VMEM((_CH, _D), jnp.float32),
            pltpu.VMEM((_CH, _D), jnp.float32),
            pltpu.VMEM((2 * _NCHN, _D), jnp.float32),
            pltpu.VMEM((_N_PAD,), jnp.float32),
            pltpu.VMEM((_N_PAD,), jnp.float32),
            pltpu.VMEM((_N_PAD,), jnp.float32),
            pltpu.VMEM((_B_PER_W * 16,), jnp.float32),
            pltpu.SemaphoreType.DMA,
            pltpu.SemaphoreType.DMA,
        ],
        compiler_params=pltpu.CompilerParams(needs_layout_passes=False),
    )
    def k(feats_hbm, px_hbm, py_hbm, pz_hbm, idx_hbm, dst_hbm, x_hbm,
          idx_v, dst_v, rows_a, rows_b, out_a, px_v, py_v, pz_v,
          sh_v, gsem_a, gsem_b):
        wid = lax.axis_index("s") * nc + lax.axis_index("c")
        ebase = wid * _B_PER_W
        nbase = wid * _NODES_W
        pltpu.sync_copy(idx_hbm.at[pl.ds(ebase, _B_PER_W)], idx_v)
        first_gather = pltpu.async_copy(
            feats_hbm.at[idx_v.at[pl.ds(0, _CH)]], rows_a, gsem_a)
        pltpu.sync_copy(dst_hbm.at[pl.ds(ebase, _B_PER_W)], dst_v)
        pltpu.sync_copy(px_hbm, px_v)
        pltpu.sync_copy(py_hbm, py_v)
        pltpu.sync_copy(pz_hbm, pz_v)

        inv_k = 1.0 / _K
        s3 = 1.7320508075688772

        @plsc.parallel_loop(0, _NGRP)
        def pbody(g):
            i16 = idx_v[pl.ds(g * 16, 16)]
            d16 = dst_v[pl.ds(g * 16, 16)]
            nx = plsc.load_gather(px_v, [i16])
            ny = plsc.load_gather(py_v, [i16])
            nz = plsc.load_gather(pz_v, [i16])
            ox = plsc.load_gather(px_v, [d16])
            oy = plsc.load_gather(py_v, [d16])
            oz = plsc.load_gather(pz_v, [d16])
            rx = nx - ox
            ry = ny - oy
            rz = nz - oz
            s = rx * rx + ry * ry + rz * rz + 1e-12
            ir = _rsqrt_nr(s)
            ux = rx * ir
            uy = ry * ir
            uz = rz * ir
            sh = (
                jnp.full((16,), inv_k, jnp.float32),
                ux * inv_k, uy * inv_k, uz * inv_k,
                (s3 * inv_k) * ux * uy,
                (s3 * inv_k) * uy * uz,
                inv_k * (1.5 * uz * uz - 0.5),
                (s3 * inv_k) * ux * uz,
                (s3 * 0.5 * inv_k) * (ux * ux - uy * uy),
            )
            lanes16 = lax.iota(jnp.int32, 16) * 16 + g * 256
            for b in range(_BASIS):
                plsc.store_scatter(sh_v, [lanes16 + b], sh[b])

        def reduce_chunk(c, buf, stage):
            @plsc.parallel_loop(0, _NCHN)
            def node_body(nl):
                e0 = nl * _K
                ebase_c = c * _CH + e0
                shrows = [sh_v[pl.ds((ebase_c + kk) * 16, 16)]
                          for kk in range(_K)]
                for b in range(_BASIS):
                    col = b * _C
                    acc = [None] * 8
                    for kk in range(_K):
                        w = shrows[kk][b]
                        for j in range(8):
                            r = buf[e0 + kk, pl.ds(col + j * 16, 16)]
                            acc[j] = w * r if kk == 0 else acc[j] + w * r
                    for j in range(8):
                        out_a[stage * _NCHN + nl,
                              pl.ds(col + j * 16, 16)] = acc[j]

        def gather_into(c, buf, sem):
            idx_c = idx_v.at[pl.ds(c * _CH, _CH)]
            return pltpu.async_copy(feats_hbm.at[idx_c], buf, sem)

        first_gather.wait()

        def chunk_body(c, carry):
            nxt = lax.rem(c + 1, _NCH)

            @pl.when(lax.rem(c, 2) == 0)
            def _():
                d = gather_into(nxt, rows_b, gsem_b)
                reduce_chunk(c, rows_a, 0)
                d.wait()

            @pl.when(lax.rem(c, 2) == 1)
            def _():
                d = gather_into(nxt, rows_a, gsem_a)
                reduce_chunk(c, rows_b, 1)
                off = pl.multiple_of(nbase + (c - 1) * _NCHN, 2 * _NCHN)
                pltpu.sync_copy(out_a, x_hbm.at[pl.ds(off, 2 * _NCHN)])
                d.wait()

            return carry

        lax.fori_loop(0, _NCH, chunk_body, 0)

    return k(feats2d, px, py, pz, idx_pad, dst_pad)


def _tc_body(x_ref, sf_ref, wcg_ref, lnw_ref, lnb_ref, w1t_ref, wgt_ref,
             w2t_ref, out_ref):
    wcg = wcg_ref[...]
    xs = []
    for b in range(_BASIS):
        xb = jnp.dot(x_ref[:, pl.ds(b * _C, _C)], wcg,
                     preferred_element_type=jnp.float32)
        xs.append(xb + sf_ref[:, pl.ds(b * _C, _C)])

    eps = 1e-5
    ys = [None] * _BASIS
    mu = jnp.mean(xs[0], axis=1, keepdims=True)
    bc = xs[0] - mu
    var0 = jnp.mean(bc * bc, axis=1, keepdims=True)
    ys[0] = bc * lax.rsqrt(var0 + eps) * lnw_ref[0:1, :] + lnb_ref[0:1, :]
    var1 = (jnp.mean(xs[1] * xs[1], axis=1, keepdims=True)
            + jnp.mean(xs[2] * xs[2], axis=1, keepdims=True)
            + jnp.mean(xs[3] * xs[3], axis=1, keepdims=True)) * (1.0 / 3.0)
    sc1 = lax.rsqrt(var1 + eps)
    for b in range(1, 4):
        ys[b] = xs[b] * sc1 * lnw_ref[1:2, :]
    var2 = (jnp.mean(xs[4] * xs[4], axis=1, keepdims=True)
            + jnp.mean(xs[5] * xs[5], axis=1, keepdims=True)
            + jnp.mean(xs[6] * xs[6], axis=1, keepdims=True)
            + jnp.mean(xs[7] * xs[7], axis=1, keepdims=True)
            + jnp.mean(xs[8] * xs[8], axis=1, keepdims=True)) * 0.2
    sc2 = lax.rsqrt(var2 + eps)
    for b in range(4, 9):
        ys[b] = xs[b] * sc2 * lnw_ref[2:3, :]

    w1t = w1t_ref[...]
    hs = [jnp.dot(ys[b], w1t, preferred_element_type=jnp.float32)
          for b in range(_BASIS)]

    after = jnp.dot(hs[0], wgt_ref[...], preferred_element_type=jnp.float32)
    type0 = after[:, 0:_MID]
    mult = jax.nn.sigmoid(after[:, _MID:])
    gs = [type0 * jax.nn.sigmoid(type0)]
    for b in range(1, 4):
        gs.append(hs[b] * mult[:, 0:_MID])
    for b in range(4, 9):
        gs.append(hs[b] * mult[:, _MID:2 * _MID])

    w2t = w2t_ref[...]
    for b in range(_BASIS):
        ob = jnp.dot(gs[b], w2t, preferred_element_type=jnp.float32)
        out_ref[:, pl.ds(b * _C, _C)] = ob + xs[b]


def _tc_dense(xsum, feats2d, W_cg, lnw, lnb, W1t, Wgt, W2t):
    return pl.pallas_call(
        _tc_body,
        grid=(_GRID,),
        in_specs=[
            pl.BlockSpec((_NB, _D), lambda i: (i, 0)),
            pl.BlockSpec((_NB, _D), lambda i: (i, 0)),
            pl.BlockSpec((_C, _C), lambda i: (0, 0)),
            pl.BlockSpec((_LMAX + 1, _C), lambda i: (0, 0)),
            pl.BlockSpec((1, _C), lambda i: (0, 0)),
            pl.BlockSpec((_C, _MID), lambda i: (0, 0)),
            pl.BlockSpec((_MID, _MID * (_LMAX + 1)), lambda i: (0, 0)),
            pl.BlockSpec((_MID, _C), lambda i: (0, 0)),
        ],
        out_specs=pl.BlockSpec((_NB, _D), lambda i: (i, 0)),
        out_shape=jax.ShapeDtypeStruct((_N, _D), jnp.float32),
        compiler_params=pltpu.CompilerParams(
            dimension_semantics=("arbitrary",),
        ),
    )(xsum, feats2d, W_cg, lnw, lnb, W1t, Wgt, W2t)


def kernel(s_feats, s_points, neighbor_indices, W_cg, ln_weight, ln_bias,
           W1, Wg, W2):
    feats2d = s_feats.reshape(_N, _D)
    ppad = jnp.pad(s_points, ((0, _N_PAD - _N), (0, 0)))
    idx_pad = jnp.pad(neighbor_indices.reshape(-1).astype(jnp.int32),
                      (0, _E_PAD - _E))
    dst_pad = jnp.repeat(
        lax.iota(jnp.int32, _N_PAD), _K, total_repeat_length=_E_PAD)

    xsum = _sc_gather_reduce(feats2d, ppad[:, 0], ppad[:, 1], ppad[:, 2],
                             idx_pad, dst_pad)

    out = _tc_dense(xsum, feats2d, W_cg, ln_weight,
                    ln_bias.reshape(1, _C), W1.T, Wg.T, W2.T)
    return out.reshape(_N, _BASIS, _C)
